# baseline (speedup 1.0000x reference)
import functools

import jax
import jax.numpy as jnp
from jax import lax
from jax.experimental import layout as jlayout
from jax.experimental import pallas as pl
from jax.experimental.pallas import tpu as pltpu
from jax.experimental.pallas import tpu_sc as plsc

CHUNK = 128
K = 5


@functools.lru_cache(maxsize=None)
def _build(B, V, D):
    info = plsc.get_sparse_core_info()
    NC, NS = info.num_cores, info.num_subcores
    NW = NC * NS
    assert B % (NW * CHUNK) == 0
    b_per_w = B // NW
    n_chunks = b_per_w // CHUNK
    n_groups = n_chunks // K
    assert n_chunks % K == 0 and n_groups >= 2
    mesh = plsc.VectorSubcoreMesh(core_axis_name="c", subcore_axis_name="s")

    @functools.partial(
        pl.kernel,
        mesh=mesh,
        out_type=jax.ShapeDtypeStruct((B, D), jnp.float32),
        compiler_params=pltpu.CompilerParams(use_tc_tiling_on_sc=False),
        scratch_types=[
            pltpu.VMEM((n_chunks, CHUNK), jnp.int32),
            pltpu.VMEM((2, K, CHUNK, D), jnp.float32),
            pltpu.SemaphoreType.DMA,
            pltpu.SemaphoreType.DMA,
            pltpu.SemaphoreType.DMA,
            pltpu.SemaphoreType.DMA,
        ],
    )
    def k(idx_hbm, table_hbm, out_hbm, idx_v, bufs, sg0, sg1, sw0, sw1):
        wid = lax.axis_index("s") * NC + lax.axis_index("c")
        base = wid * b_per_w
        pltpu.sync_copy(idx_hbm.at[wid], idx_v)
        sems_g = (sg0, sg1)
        sems_w = (sw0, sw1)

        def fire_gathers(g, s):
            for b in range(K):
                pltpu.async_copy(
                    table_hbm.at[idx_v.at[g * K + b]], bufs.at[s].at[b], sems_g[s]
                )

        def drain_gathers(g, s):
            for b in range(K):
                pltpu.make_async_copy(
                    table_hbm.at[idx_v.at[g * K + b]], bufs.at[s].at[b], sems_g[s]
                ).wait()

        def fire_writes(g, s):
            for b in range(K):
                pltpu.async_copy(
                    bufs.at[s].at[b],
                    out_hbm.at[pl.ds(base + (g * K + b) * CHUNK, CHUNK)],
                    sems_w[s],
                )

        def drain_writes(g, s):
            for b in range(K):
                pltpu.make_async_copy(
                    bufs.at[s].at[b],
                    out_hbm.at[pl.ds(base + (g * K + b) * CHUNK, CHUNK)],
                    sems_w[s],
                ).wait()

        fire_gathers(0, 0)

        def step(g, carry):
            ga = 2 * g
            gb = 2 * g + 1

            @pl.when(g > 0)
            def _():
                drain_writes(ga - 1, 1)

            fire_gathers(gb, 1)
            drain_gathers(ga, 0)
            fire_writes(ga, 0)

            @pl.when(gb + 1 < n_groups)
            def _():
                drain_writes(ga, 0)
                fire_gathers(gb + 1, 0)

            drain_gathers(gb, 1)
            fire_writes(gb, 1)
            return carry

        lax.fori_loop(0, n_groups // 2, step, 0)

        if n_groups % 2:
            t = n_groups - 1
            drain_gathers(t, 0)
            fire_writes(t, 0)
            drain_writes(t - 1, 1)
            drain_writes(t, 0)
        else:
            drain_writes(n_groups - 2, 0)
            drain_writes(n_groups - 1, 1)

    return k


def kernel(idx, table):
    bsz, slen = idx.shape
    V, D = table.shape
    info = plsc.get_sparse_core_info()
    nw = info.num_cores * info.num_subcores
    half = slen // 2
    fn = _build(bsz * half, V, D)
    outs = []
    for sl in (idx[:, :half], idx[:, half:]):
        B = bsz * half
        idx3d = sl.reshape(nw, B // (nw * CHUNK), CHUNK)
        outs.append(fn(idx3d, table).reshape(bsz, half, D))
    return jnp.concatenate(outs, axis=1)


# speedup vs baseline: 1.1093x; 1.1093x over previous
import functools

import jax
import jax.numpy as jnp
from jax import lax
from jax.experimental import pallas as pl
from jax.experimental.pallas import tpu as pltpu
from jax.experimental.pallas import tpu_sc as plsc

CHUNK = 128
K = 5


@functools.lru_cache(maxsize=None)
def _build(B, V, D):
    info = plsc.get_sparse_core_info()
    NC, NS = info.num_cores, info.num_subcores
    NW = NC * NS
    assert B % (NW * CHUNK) == 0
    b_per_w = B // NW
    n_chunks = b_per_w // CHUNK
    n_groups = n_chunks // K
    assert n_chunks % K == 0 and n_groups >= 2
    mesh = plsc.VectorSubcoreMesh(core_axis_name="c", subcore_axis_name="s")

    @functools.partial(
        pl.kernel,
        mesh=mesh,
        out_type=jax.ShapeDtypeStruct((B, D), jnp.float32),
        compiler_params=pltpu.CompilerParams(use_tc_tiling_on_sc=False),
        scratch_types=[
            pltpu.VMEM((n_chunks, CHUNK), jnp.int32),
            pltpu.VMEM((2, K, CHUNK, D), jnp.float32),
            pltpu.SemaphoreType.DMA,
            pltpu.SemaphoreType.DMA,
            pltpu.SemaphoreType.DMA,
            pltpu.SemaphoreType.DMA,
        ],
    )
    def k(idx_hbm, table_hbm, out_hbm, idx_v, bufs, sg0, sg1, sw0, sw1):
        wid = lax.axis_index("s") * NC + lax.axis_index("c")
        base = wid * b_per_w
        pltpu.sync_copy(idx_hbm.at[wid], idx_v)
        sems_g = (sg0, sg1)
        sems_w = (sw0, sw1)

        def fire_gathers(g, s):
            for b in range(K):
                pltpu.async_copy(
                    table_hbm.at[idx_v.at[g * K + b]], bufs.at[s].at[b], sems_g[s]
                )

        def drain_gathers(g, s):
            for b in range(K):
                pltpu.make_async_copy(
                    table_hbm.at[idx_v.at[g * K + b]], bufs.at[s].at[b], sems_g[s]
                ).wait()

        def fire_writes(g, s):
            for b in range(K):
                pltpu.async_copy(
                    bufs.at[s].at[b],
                    out_hbm.at[pl.ds(base + (g * K + b) * CHUNK, CHUNK)],
                    sems_w[s],
                )

        def drain_writes(g, s):
            for b in range(K):
                pltpu.make_async_copy(
                    bufs.at[s].at[b],
                    out_hbm.at[pl.ds(base + (g * K + b) * CHUNK, CHUNK)],
                    sems_w[s],
                ).wait()

        fire_gathers(0, 0)

        def step(g, carry):
            ga = 2 * g
            gb = 2 * g + 1

            @pl.when(g > 0)
            def _():
                drain_writes(ga - 1, 1)

            fire_gathers(gb, 1)
            drain_gathers(ga, 0)
            fire_writes(ga, 0)

            @pl.when(gb + 1 < n_groups)
            def _():
                drain_writes(ga, 0)
                fire_gathers(gb + 1, 0)

            drain_gathers(gb, 1)
            fire_writes(gb, 1)
            return carry

        lax.fori_loop(0, n_groups // 2, step, 0)

        if n_groups % 2:
            t = n_groups - 1
            drain_gathers(t, 0)
            fire_writes(t, 0)
            drain_writes(t - 1, 1)
            drain_writes(t, 0)
        else:
            drain_writes(n_groups - 2, 0)
            drain_writes(n_groups - 1, 1)

    return k


def kernel(idx, table):
    bsz, slen = idx.shape
    V, D = table.shape
    B = bsz * slen
    info = plsc.get_sparse_core_info()
    nw = info.num_cores * info.num_subcores
    idx3d = idx.reshape(nw, B // (nw * CHUNK), CHUNK)
    out = _build(B, V, D)(idx3d, table)
    return out.reshape(bsz, slen, D)
